# baseline (speedup 1.0000x reference)
import functools

import jax
import jax.numpy as jnp
from jax import lax
from jax.experimental import pallas as pl
from jax.experimental.pallas import tpu as pltpu
from jax.experimental.pallas import tpu_sc as plsc

NUM_FRAGMENTS = 128
MAX_ATTACH = 16
HIDDEN = 128
BATCH = 16384

NUM_CORES = 2
NUM_SUBCORES = 16
NUM_WORKERS = NUM_CORES * NUM_SUBCORES
BPW = BATCH // NUM_WORKERS
G = 16
BLK = MAX_ATTACH * HIDDEN
GBLK = G * BLK
TABW = (NUM_FRAGMENTS + MAX_ATTACH) * HIDDEN


@jax.jit
def _fragment_gather(fragment_idx, attach_table, attach_mask):
  mesh = plsc.VectorSubcoreMesh(core_axis_name="c", subcore_axis_name="s")

  @functools.partial(
      pl.kernel,
      out_type=(
          jax.ShapeDtypeStruct((BATCH * MAX_ATTACH * HIDDEN,), jnp.float32),
          jax.ShapeDtypeStruct((BATCH * MAX_ATTACH,), jnp.float32),
      ),
      mesh=mesh,
      scratch_types=[
          pltpu.VMEM((TABW,), jnp.float32),
          pltpu.VMEM((BPW,), jnp.int32),
          pltpu.VMEM((NUM_FRAGMENTS * MAX_ATTACH,), jnp.float32),
          pltpu.VMEM((BPW * MAX_ATTACH,), jnp.float32),
          pltpu.SemaphoreType.DMA,
          pltpu.SemaphoreType.DMA,
          pltpu.SemaphoreType.DMA,
          pltpu.SemaphoreType.DMA,
          pltpu.SemaphoreType.DMA,
          pltpu.SemaphoreType.DMA,
          pltpu.SemaphoreType.DMA,
          pltpu.SemaphoreType.DMA,
          pltpu.SemaphoreType.DMA,
      ],
  )
  def k(fi_hbm, tab_hbm, msk_hbm, oemb, omsk, tbuf, fi_v, mvmem, mout,
        outsem0, outsem1, outsem2, outsem3, outsem4, outsem5, outsem6,
        outsem7, auxsem):
    outsems = (outsem0, outsem1, outsem2, outsem3, outsem4, outsem5, outsem6,
               outsem7)
    wid = lax.axis_index("s") * NUM_CORES + lax.axis_index("c")
    base = wid * BPW
    pltpu.sync_copy(fi_hbm.at[pl.ds(base, BPW)], fi_v)
    pltpu.async_copy(tab_hbm.at[pl.ds(0, TABW)], tbuf, auxsem)
    pltpu.sync_copy(msk_hbm, mvmem)
    pltpu.make_async_copy(tab_hbm.at[pl.ds(0, TABW)], tbuf, auxsem).wait()

    @pl.loop(0, BPW, step=G)
    def _(b0):
      svec = fi_v[pl.ds(b0, G)] * HIDDEN
      dstb = (base + b0) * BLK
      for t in range(G):
        pltpu.async_copy(tbuf.at[pl.ds(pl.multiple_of(svec[t], HIDDEN), BLK)],
                         oemb.at[pl.ds(dstb + t * BLK, BLK)], outsems[t % 8])

    @pl.loop(0, BPW, step=G)
    def _(b0):
      fvec = fi_v[pl.ds(b0, G)]
      for t in range(G):
        mout[pl.ds((b0 + t) * MAX_ATTACH, MAX_ATTACH)] = (
            mvmem[pl.ds(fvec[t] * MAX_ATTACH, MAX_ATTACH)])

    @pl.loop(0, BPW, step=G)
    def _(b0):
      for s in range(8):
        pltpu.make_async_copy(tab_hbm.at[pl.ds(0, GBLK // 8)],
                              oemb.at[pl.ds(0, GBLK // 8)], outsems[s]).wait()

    pltpu.async_copy(mout, omsk.at[pl.ds(base * MAX_ATTACH, BPW * MAX_ATTACH)],
                     auxsem).wait()

  return k(fragment_idx, attach_table.reshape(-1), attach_mask.reshape(-1))


def kernel(fragment_idx, attach_table, attach_mask):
  fi = fragment_idx
  if fi.ndim == 0:
    fi = fi[None]
  fi = fi.astype(jnp.int32)
  emb_flat, mask_flat = _fragment_gather(fi, attach_table, attach_mask)
  emb = emb_flat.reshape(BATCH, MAX_ATTACH, HIDDEN)
  return emb, mask_flat.reshape(BATCH, MAX_ATTACH)
